# baseline (speedup 1.0000x reference)
import functools

import jax
import jax.numpy as jnp
from jax import lax
from jax.experimental import pallas as pl
from jax.experimental.pallas import tpu as pltpu
from jax.experimental.pallas import tpu_sc as plsc

N_TOK = 8192
DIM = 256
N_EMB = 8192

BM = 256
BN = 512
GRID_R = N_TOK // BM
GRID_C = N_EMB // BN

SC_CORES = 2
SC_SUBCORES = 16
SC_WORKERS = SC_CORES * SC_SUBCORES
ROWS_PER_WORKER = N_TOK // SC_WORKERS


def _dist_argmin_body(x_ref, e_ref, idx_ref, et_ref, bd_ref, bi_ref):
    c = pl.program_id(0)
    r = pl.program_id(1)
    rows = pl.ds(r * BM, BM)

    x_t = x_ref[rows, :]
    e_t = e_ref[...]
    sim = jnp.dot(x_t, e_t, preferred_element_type=jnp.float32)
    x2 = jnp.sum(x_t * x_t, axis=1, keepdims=True)
    e2 = jnp.sum(e_t * e_t, axis=0, keepdims=True)
    d = (x2 + e2) - 2.0 * sim

    lmin = jnp.min(d, axis=1, keepdims=True)
    col = jax.lax.broadcasted_iota(jnp.int32, d.shape, 1) + c * BN
    larg = jnp.min(jnp.where(d == lmin, col, jnp.int32(2**31 - 1)),
                   axis=1, keepdims=True)

    prev_d = bd_ref[rows, :]
    prev_i = bi_ref[rows, :]
    upd = jnp.logical_or(c == 0, lmin < prev_d)
    new_d = jnp.where(upd, lmin, prev_d)
    new_i = jnp.where(upd, larg, prev_i)
    bd_ref[rows, :] = new_d
    bi_ref[rows, :] = new_i
    idx_ref[...] = new_i

    @pl.when(r == 0)
    def _():
        et_ref[...] = e_t.T


def _dist_argmin(flat, embeddings):
    return pl.pallas_call(
        _dist_argmin_body,
        grid=(GRID_C, GRID_R),
        in_specs=[
            pl.BlockSpec((N_TOK, DIM), lambda c, r: (0, 0)),
            pl.BlockSpec((DIM, BN), lambda c, r: (0, c)),
        ],
        out_specs=[
            pl.BlockSpec((BM, 1), lambda c, r: (r, 0)),
            pl.BlockSpec((BN, DIM), lambda c, r: (c, 0)),
        ],
        out_shape=[
            jax.ShapeDtypeStruct((N_TOK, 1), jnp.int32),
            jax.ShapeDtypeStruct((N_EMB, DIM), jnp.float32),
        ],
        scratch_shapes=[
            pltpu.VMEM((N_TOK, 1), jnp.float32),
            pltpu.VMEM((N_TOK, 1), jnp.int32),
        ],
        compiler_params=pltpu.CompilerParams(
            dimension_semantics=("arbitrary", "arbitrary"),
        ),
    )(flat, embeddings)


def _sc_gather_body(table_hbm, idx_hbm, out_hbm, idx_v, rows_v, sem):
    wid = lax.axis_index("s") * SC_CORES + lax.axis_index("c")
    base = wid * ROWS_PER_WORKER
    pltpu.sync_copy(idx_hbm.at[pl.ds(base, ROWS_PER_WORKER)], idx_v)
    pltpu.async_copy(table_hbm.at[idx_v], rows_v, sem).wait()
    pltpu.sync_copy(rows_v, out_hbm.at[pl.ds(base, ROWS_PER_WORKER)])


def _sc_gather(table, idx):
    f = functools.partial(
        pl.kernel,
        out_type=jax.ShapeDtypeStruct((N_TOK, DIM), jnp.float32),
        mesh=plsc.VectorSubcoreMesh(core_axis_name="c", subcore_axis_name="s"),
        scratch_types=[
            pltpu.VMEM((ROWS_PER_WORKER,), jnp.int32),
            pltpu.VMEM((ROWS_PER_WORKER, DIM), jnp.float32),
            pltpu.SemaphoreType.DMA,
        ],
    )(_sc_gather_body)
    return f(table, idx)


def kernel(x, embeddings):
    flat = x.reshape(N_TOK, DIM)
    idx2, e_t = _dist_argmin(flat, embeddings)
    quantized = _sc_gather(e_t, idx2.reshape(N_TOK))
    return quantized.reshape(x.shape)


# speedup vs baseline: 2.4932x; 2.4932x over previous
import functools

import jax
import jax.numpy as jnp
from jax import lax
from jax.experimental import pallas as pl
from jax.experimental.pallas import tpu as pltpu
from jax.experimental.pallas import tpu_sc as plsc

N_TOK = 8192
DIM = 256
N_EMB = 8192

BM = 256
GRID_R = N_TOK // BM

SC_CORES = 2
SC_SUBCORES = 16
SC_WORKERS = SC_CORES * SC_SUBCORES
ROWS_PER_WORKER = N_TOK // SC_WORKERS


def _enorms_body(e_ref, e2_ref):
    e = e_ref[...]
    e2_ref[...] = jnp.sum(e * e, axis=0, keepdims=True)


def _enorms(embeddings):
    return pl.pallas_call(
        _enorms_body,
        out_shape=jax.ShapeDtypeStruct((1, N_EMB), jnp.float32),
    )(embeddings)


def _dist_argmin_body(x_ref, e_ref, e2_ref, idx_ref):
    x_t = x_ref[...]
    e_t = e_ref[...]
    sim = jnp.dot(x_t, e_t, preferred_element_type=jnp.float32)
    x2 = jnp.sum(x_t * x_t, axis=1, keepdims=True)
    e2 = e2_ref[...]
    d = (x2 + e2) - 2.0 * sim
    lmin = jnp.min(d, axis=1, keepdims=True)
    col = jax.lax.broadcasted_iota(jnp.int32, d.shape, 1)
    larg = jnp.min(jnp.where(d == lmin, col, jnp.int32(2**31 - 1)),
                   axis=1, keepdims=True)
    idx_ref[...] = larg


def _dist_argmin(flat, embeddings, e2):
    return pl.pallas_call(
        _dist_argmin_body,
        grid=(GRID_R,),
        in_specs=[
            pl.BlockSpec((BM, DIM), lambda r: (r, 0)),
            pl.BlockSpec((DIM, N_EMB), lambda r: (0, 0)),
            pl.BlockSpec((1, N_EMB), lambda r: (0, 0)),
        ],
        out_specs=pl.BlockSpec((BM, 1), lambda r: (r, 0)),
        out_shape=jax.ShapeDtypeStruct((N_TOK, 1), jnp.int32),
        compiler_params=pltpu.CompilerParams(
            dimension_semantics=("arbitrary",),
        ),
    )(flat, embeddings, e2)


TBN = 512


def _transpose_body(e_ref, et_ref):
    et_ref[...] = e_ref[...].T


def _transpose(embeddings):
    return pl.pallas_call(
        _transpose_body,
        grid=(N_EMB // TBN,),
        in_specs=[pl.BlockSpec((DIM, TBN), lambda i: (0, i))],
        out_specs=pl.BlockSpec((TBN, DIM), lambda i: (i, 0)),
        out_shape=jax.ShapeDtypeStruct((N_EMB, DIM), jnp.float32),
    )(embeddings)


def _sc_gather_body(table_hbm, idx_hbm, out_hbm, idx_v, rows_v, sem):
    wid = lax.axis_index("s") * SC_CORES + lax.axis_index("c")
    base = wid * ROWS_PER_WORKER
    pltpu.sync_copy(idx_hbm.at[pl.ds(base, ROWS_PER_WORKER)], idx_v)
    pltpu.async_copy(table_hbm.at[idx_v], rows_v, sem).wait()
    pltpu.sync_copy(rows_v, out_hbm.at[pl.ds(base, ROWS_PER_WORKER)])


def _sc_gather(table, idx):
    f = functools.partial(
        pl.kernel,
        out_type=jax.ShapeDtypeStruct((N_TOK, DIM), jnp.float32),
        mesh=plsc.VectorSubcoreMesh(core_axis_name="c", subcore_axis_name="s"),
        scratch_types=[
            pltpu.VMEM((ROWS_PER_WORKER,), jnp.int32),
            pltpu.VMEM((ROWS_PER_WORKER, DIM), jnp.float32),
            pltpu.SemaphoreType.DMA,
        ],
    )(_sc_gather_body)
    return f(table, idx)


def kernel(x, embeddings):
    flat = x.reshape(N_TOK, DIM)
    e2 = _enorms(embeddings)
    idx2 = _dist_argmin(flat, embeddings, e2)
    e_t = _transpose(embeddings)
    quantized = _sc_gather(e_t, idx2.reshape(N_TOK))
    return quantized.reshape(x.shape)


# speedup vs baseline: 2.5909x; 1.0392x over previous
import functools

import jax
import jax.numpy as jnp
from jax import lax
from jax.experimental import pallas as pl
from jax.experimental.pallas import tpu as pltpu
from jax.experimental.pallas import tpu_sc as plsc

N_TOK = 8192
DIM = 256
N_EMB = 8192

BM = 256
GRID_R = N_TOK // BM

SC_CORES = 2
SC_SUBCORES = 16
SC_WORKERS = SC_CORES * SC_SUBCORES
ROWS_PER_WORKER = N_TOK // SC_WORKERS


TBN = 512


def _prep_body(e_ref, e2_ref, et_ref):
    e = e_ref[...]
    e2_ref[...] = jnp.sum(e * e, axis=0, keepdims=True)
    et_ref[...] = e.T


def _prep(embeddings):
    return pl.pallas_call(
        _prep_body,
        grid=(N_EMB // TBN,),
        in_specs=[pl.BlockSpec((DIM, TBN), lambda i: (0, i))],
        out_specs=[
            pl.BlockSpec((1, TBN), lambda i: (0, i)),
            pl.BlockSpec((TBN, DIM), lambda i: (i, 0)),
        ],
        out_shape=[
            jax.ShapeDtypeStruct((1, N_EMB), jnp.float32),
            jax.ShapeDtypeStruct((N_EMB, DIM), jnp.float32),
        ],
    )(embeddings)


def _dist_argmin_body(x_ref, e_ref, e2_ref, idx_ref):
    x_t = x_ref[...]
    e_t = e_ref[...]
    sim = jnp.dot(x_t, e_t, preferred_element_type=jnp.float32)
    x2 = jnp.sum(x_t * x_t, axis=1, keepdims=True)
    e2 = e2_ref[...]
    d = (x2 + e2) - 2.0 * sim
    lmin = jnp.min(d, axis=1, keepdims=True)
    col = jax.lax.broadcasted_iota(jnp.int32, d.shape, 1)
    larg = jnp.min(jnp.where(d == lmin, col, jnp.int32(2**31 - 1)),
                   axis=1, keepdims=True)
    idx_ref[...] = larg


def _dist_argmin(flat, embeddings, e2):
    return pl.pallas_call(
        _dist_argmin_body,
        grid=(GRID_R,),
        in_specs=[
            pl.BlockSpec((BM, DIM), lambda r: (r, 0)),
            pl.BlockSpec((DIM, N_EMB), lambda r: (0, 0)),
            pl.BlockSpec((1, N_EMB), lambda r: (0, 0)),
        ],
        out_specs=pl.BlockSpec((BM, 1), lambda r: (r, 0)),
        out_shape=jax.ShapeDtypeStruct((N_TOK, 1), jnp.int32),
        compiler_params=pltpu.CompilerParams(
            dimension_semantics=("arbitrary",),
        ),
    )(flat, embeddings, e2)


def _sc_gather_body(table_hbm, idx_hbm, out_hbm, idx_v, rows_v, sem):
    wid = lax.axis_index("s") * SC_CORES + lax.axis_index("c")
    base = wid * ROWS_PER_WORKER
    pltpu.sync_copy(idx_hbm.at[pl.ds(base, ROWS_PER_WORKER)], idx_v)
    pltpu.async_copy(table_hbm.at[idx_v], rows_v, sem).wait()
    pltpu.sync_copy(rows_v, out_hbm.at[pl.ds(base, ROWS_PER_WORKER)])


def _sc_gather(table, idx):
    f = functools.partial(
        pl.kernel,
        out_type=jax.ShapeDtypeStruct((N_TOK, DIM), jnp.float32),
        mesh=plsc.VectorSubcoreMesh(core_axis_name="c", subcore_axis_name="s"),
        scratch_types=[
            pltpu.VMEM((ROWS_PER_WORKER,), jnp.int32),
            pltpu.VMEM((ROWS_PER_WORKER, DIM), jnp.float32),
            pltpu.SemaphoreType.DMA,
        ],
    )(_sc_gather_body)
    return f(table, idx)


def kernel(x, embeddings):
    flat = x.reshape(N_TOK, DIM)
    e2, e_t = _prep(embeddings)
    idx2 = _dist_argmin(flat, embeddings, e2)
    quantized = _sc_gather(e_t, idx2.reshape(N_TOK))
    return quantized.reshape(x.shape)


# speedup vs baseline: 2.6131x; 1.0086x over previous
import functools

import jax
import jax.numpy as jnp
from jax import lax
from jax.experimental import pallas as pl
from jax.experimental.pallas import tpu as pltpu
from jax.experimental.pallas import tpu_sc as plsc

N_TOK = 8192
DIM = 256
N_EMB = 8192

BM = 256
GRID_R = N_TOK // BM

SC_CORES = 2
SC_SUBCORES = 16
SC_WORKERS = SC_CORES * SC_SUBCORES
ROWS_PER_WORKER = N_TOK // SC_WORKERS


TBN = 512


def _prep_body(e_ref, e2_ref, et_ref):
    e = e_ref[...]
    e2_ref[...] = jnp.sum(e * e, axis=0, keepdims=True)
    et_ref[...] = e.T


def _prep(embeddings):
    return pl.pallas_call(
        _prep_body,
        grid=(N_EMB // TBN,),
        in_specs=[pl.BlockSpec((DIM, TBN), lambda i: (0, i))],
        out_specs=[
            pl.BlockSpec((1, TBN), lambda i: (0, i)),
            pl.BlockSpec((TBN, DIM), lambda i: (i, 0)),
        ],
        out_shape=[
            jax.ShapeDtypeStruct((1, N_EMB), jnp.float32),
            jax.ShapeDtypeStruct((N_EMB, DIM), jnp.float32),
        ],
    )(embeddings)


def _dist_argmin_body(x_ref, e_ref, e2_ref, idx_ref):
    x_t = x_ref[...]
    e_t = e_ref[...]
    sim = jnp.dot(x_t, e_t, preferred_element_type=jnp.float32)
    x2 = jnp.sum(x_t * x_t, axis=1, keepdims=True)
    e2 = e2_ref[...]
    d = (x2 + e2) - 2.0 * sim
    lmin = jnp.min(d, axis=1, keepdims=True)
    col = jax.lax.broadcasted_iota(jnp.int32, d.shape, 1)
    larg = jnp.min(jnp.where(d == lmin, col, jnp.int32(2**31 - 1)),
                   axis=1)
    idx_ref[...] = larg


def _dist_argmin(flat, embeddings, e2):
    return pl.pallas_call(
        _dist_argmin_body,
        grid=(GRID_R,),
        in_specs=[
            pl.BlockSpec((BM, DIM), lambda r: (r, 0)),
            pl.BlockSpec((DIM, N_EMB), lambda r: (0, 0)),
            pl.BlockSpec((1, N_EMB), lambda r: (0, 0)),
        ],
        out_specs=pl.BlockSpec((BM,), lambda r: (r,)),
        out_shape=jax.ShapeDtypeStruct((N_TOK,), jnp.int32),
        compiler_params=pltpu.CompilerParams(
            dimension_semantics=("arbitrary",),
        ),
    )(flat, embeddings, e2)


def _sc_gather_body(table_hbm, idx_hbm, out_hbm, idx_v, rows_v, sem):
    wid = lax.axis_index("s") * SC_CORES + lax.axis_index("c")
    base = wid * ROWS_PER_WORKER
    pltpu.sync_copy(idx_hbm.at[pl.ds(base, ROWS_PER_WORKER)], idx_v)
    pltpu.async_copy(table_hbm.at[idx_v], rows_v, sem).wait()
    pltpu.sync_copy(rows_v, out_hbm.at[pl.ds(base, ROWS_PER_WORKER)])


def _sc_gather(table, idx):
    f = functools.partial(
        pl.kernel,
        out_type=jax.ShapeDtypeStruct((N_TOK, DIM), jnp.float32),
        mesh=plsc.VectorSubcoreMesh(core_axis_name="c", subcore_axis_name="s"),
        scratch_types=[
            pltpu.VMEM((ROWS_PER_WORKER,), jnp.int32),
            pltpu.VMEM((ROWS_PER_WORKER, DIM), jnp.float32),
            pltpu.SemaphoreType.DMA,
        ],
    )(_sc_gather_body)
    return f(table, idx)


def kernel(x, embeddings):
    flat = x.reshape(N_TOK, DIM)
    e2, e_t = _prep(embeddings)
    idx = _dist_argmin(flat, embeddings, e2)
    quantized = _sc_gather(e_t, idx)
    return quantized.reshape(x.shape)


# speedup vs baseline: 2.8027x; 1.0725x over previous
import functools

import jax
import jax.numpy as jnp
from jax import lax
from jax.experimental import pallas as pl
from jax.experimental.pallas import tpu as pltpu
from jax.experimental.pallas import tpu_sc as plsc

N_TOK = 8192
DIM = 256
N_EMB = 8192

BM = 256
GRID_R = N_TOK // BM

SC_CORES = 2
SC_SUBCORES = 16
SC_WORKERS = SC_CORES * SC_SUBCORES
ROWS_PER_WORKER = N_TOK // SC_WORKERS


TBN = 512


def _prep_body(e_ref, e2_ref, et_ref):
    e = e_ref[...]
    e2_ref[...] = jnp.sum(e * e, axis=0, keepdims=True)
    et_ref[...] = e.T


def _prep(embeddings):
    return pl.pallas_call(
        _prep_body,
        grid=(N_EMB // TBN,),
        in_specs=[pl.BlockSpec((DIM, TBN), lambda i: (0, i))],
        out_specs=[
            pl.BlockSpec((1, TBN), lambda i: (0, i)),
            pl.BlockSpec((TBN, DIM), lambda i: (i, 0)),
        ],
        out_shape=[
            jax.ShapeDtypeStruct((1, N_EMB), jnp.float32),
            jax.ShapeDtypeStruct((N_EMB, DIM), jnp.float32),
        ],
    )(embeddings)


def _dist_argmin_body(x_ref, e_ref, e2_ref, idx_ref):
    x_t = x_ref[...]
    e_t = e_ref[...]
    sim = jnp.dot(x_t, e_t, preferred_element_type=jnp.float32)
    x2 = jnp.sum(x_t * x_t, axis=1, keepdims=True)
    e2 = e2_ref[...]
    d = (x2 + e2) - 2.0 * sim
    lmin = jnp.min(d, axis=1, keepdims=True)
    col = jax.lax.broadcasted_iota(
        jnp.int32, (1, N_EMB), 1).astype(jnp.float32)
    larg = jnp.min(jnp.where(d == lmin, col, jnp.float32(2.0**30)),
                   axis=1, keepdims=True)
    idx_ref[...] = larg.astype(jnp.int32)


def _dist_argmin(flat, embeddings, e2):
    return pl.pallas_call(
        _dist_argmin_body,
        grid=(GRID_R,),
        in_specs=[
            pl.BlockSpec((BM, DIM), lambda r: (r, 0)),
            pl.BlockSpec((DIM, N_EMB), lambda r: (0, 0)),
            pl.BlockSpec((1, N_EMB), lambda r: (0, 0)),
        ],
        out_specs=pl.BlockSpec((BM, 1), lambda r: (r, 0)),
        out_shape=jax.ShapeDtypeStruct((N_TOK, 1), jnp.int32),
        compiler_params=pltpu.CompilerParams(
            dimension_semantics=("arbitrary",),
        ),
    )(flat, embeddings, e2)


def _sc_gather_body(table_hbm, idx_hbm, out_hbm, idx_v, rows_v, sem):
    wid = lax.axis_index("s") * SC_CORES + lax.axis_index("c")
    base = wid * ROWS_PER_WORKER
    pltpu.sync_copy(idx_hbm.at[pl.ds(base, ROWS_PER_WORKER)], idx_v)
    pltpu.async_copy(table_hbm.at[idx_v], rows_v, sem).wait()
    pltpu.sync_copy(rows_v, out_hbm.at[pl.ds(base, ROWS_PER_WORKER)])


def _sc_gather(table, idx):
    f = functools.partial(
        pl.kernel,
        out_type=jax.ShapeDtypeStruct((N_TOK, DIM), jnp.float32),
        mesh=plsc.VectorSubcoreMesh(core_axis_name="c", subcore_axis_name="s"),
        scratch_types=[
            pltpu.VMEM((ROWS_PER_WORKER,), jnp.int32),
            pltpu.VMEM((ROWS_PER_WORKER, DIM), jnp.float32),
            pltpu.SemaphoreType.DMA,
        ],
    )(_sc_gather_body)
    return f(table, idx)


def kernel(x, embeddings):
    flat = x.reshape(N_TOK, DIM)
    e2, e_t = _prep(embeddings)
    idx = _dist_argmin(flat, embeddings, e2)
    quantized = _sc_gather(e_t, idx.reshape(N_TOK))
    return quantized.reshape(x.shape)
